# baseline (speedup 1.0000x reference)
import jax
import jax.numpy as jnp
from jax import lax
from jax.experimental import pallas as pl
from jax.experimental.pallas import tpu as pltpu
from jax.experimental.pallas import tpu_sc as plsc

N = 10000
D = 256
H = 128
E = 320000
Q = 65536
NC = 2
NS = 16
BNS = 1.0 / (1.0 + 1e-5) ** 0.5

RPT = 624

KD = 125
NCHD = E // (NC * NS) // KD
DW = 128
KA = 125
NCHA = E // NS // KA
PCH = 40
KQ = 128
NCHQ = Q // NS // KQ


def _per_tile_rows(s, fn):
    fn(s * RPT, RPT)

    @pl.when(s == 0)
    def _():
        fn(NS * RPT, N - NS * RPT)


def _vmesh():
    return plsc.VectorSubcoreMesh(
        core_axis_name="c", subcore_axis_name="s", num_cores=NC, num_subcores=NS)


def _deg_body(dstr_hbm, ones_hbm, zeros_hbm, deg_out, dacc, idx_v, ones_v):
    c = lax.axis_index("c")
    s = lax.axis_index("s")
    wid = c * NS + s
    _per_tile_rows(s, lambda o, n: pltpu.sync_copy(
        zeros_hbm.at[pl.ds(o, n)], dacc.at[pl.ds(o, n)]))
    pltpu.sync_copy(dstr_hbm.at[wid], idx_v)
    pltpu.sync_copy(ones_hbm, ones_v)
    plsc.subcore_barrier()

    def chunk(j, carry):
        pltpu.sync_copy(ones_v, dacc.at[idx_v.at[j]], add=True)
        return carry

    lax.fori_loop(0, NCHD, chunk, 0)
    plsc.subcore_barrier()
    _per_tile_rows(s, lambda o, n: pltpu.sync_copy(
        dacc.at[pl.ds(o, n)], deg_out.at[c, pl.ds(o, n)]))


def _sc_degree(dstr):
    f = pl.kernel(
        _deg_body,
        out_type=jax.ShapeDtypeStruct((NC, N, DW), jnp.float32),
        mesh=_vmesh(),
        scratch_types=[
            pltpu.VMEM_SHARED((N, DW), jnp.float32),
            pltpu.VMEM((NCHD, KD), jnp.int32),
            pltpu.VMEM((KD, DW), jnp.float32),
        ],
    )
    return f(dstr, jnp.ones((KD, DW), jnp.float32), jnp.zeros((N, DW), jnp.float32))


def _agg_body(hs_lo, hs_hi, srcr, dstr, agg_out, acc, idx_s, idx_d, rows,
              sem0, sem1):
    c = lax.axis_index("c")
    s = lax.axis_index("s")
    coff = c * N

    def run(hs_hbm):
        _per_tile_rows(s, lambda o, n: pltpu.sync_copy(
            hs_hbm.at[pl.ds(o, n)], acc.at[pl.ds(o, n)]))
        plsc.subcore_barrier()

        def start(j, b, sem):
            pltpu.async_copy(hs_hbm.at[idx_s.at[j]], rows.at[b], sem)

        def wait(j, b, sem):
            pltpu.make_async_copy(hs_hbm.at[idx_s.at[j]], rows.at[b], sem).wait()

        def scat(j, b):
            pltpu.sync_copy(rows.at[b], acc.at[idx_d.at[j]], add=True)

        def phase(p, carry):
            pltpu.sync_copy(srcr.at[s, pl.ds(p * PCH, PCH)], idx_s)
            pltpu.sync_copy(dstr.at[s, pl.ds(p * PCH, PCH)], idx_d)
            start(0, 0, sem0)

            def pair(m, carry2):
                j0 = m * 2
                start(j0 + 1, 1, sem1)
                wait(j0, 0, sem0)
                scat(j0, 0)

                @pl.when(j0 + 2 < PCH)
                def _():
                    start(j0 + 2, 0, sem0)

                wait(j0 + 1, 1, sem1)
                scat(j0 + 1, 1)
                return carry2

            lax.fori_loop(0, PCH // 2, pair, 0)
            return carry

        lax.fori_loop(0, NCHA // PCH, phase, 0)
        plsc.subcore_barrier()
        _per_tile_rows(s, lambda o, n: pltpu.sync_copy(
            acc.at[pl.ds(o, n)], agg_out.at[pl.ds(coff + o, n)]))

    @pl.when(c == 0)
    def _():
        run(hs_lo)

    @pl.when(c == 1)
    def _():
        run(hs_hi)


def _sc_agg(hs, srcr, dstr):
    f = pl.kernel(
        _agg_body,
        out_type=jax.ShapeDtypeStruct((NC * N, H), jnp.float32),
        mesh=_vmesh(),
        scratch_types=[
            pltpu.VMEM_SHARED((N, H), jnp.float32),
            pltpu.VMEM((PCH, KA), jnp.int32),
            pltpu.VMEM((PCH, KA), jnp.int32),
            pltpu.VMEM((2, KA, H), jnp.float32),
            pltpu.SemaphoreType.DMA,
            pltpu.SemaphoreType.DMA,
        ],
    )
    return f(hs[0], hs[1], srcr, dstr)


def _qp_body(h2_lo, h2_hi, qsr, qtr, p_out, idx_a, idx_b, ra, rb,
             sa0, sb0, sa1, sb1):
    c = lax.axis_index("c")
    s = lax.axis_index("s")
    pltpu.sync_copy(qsr.at[s], idx_a)
    pltpu.sync_copy(qtr.at[s], idx_b)
    obase = c * Q + s * (Q // NS)

    def run(h2):
        sems = ((sa0, sb0), (sa1, sb1))

        def start(j, b):
            pltpu.async_copy(h2.at[idx_a.at[j]], ra.at[b], sems[b][0])
            pltpu.async_copy(h2.at[idx_b.at[j]], rb.at[b], sems[b][1])

        def wait(j, b):
            pltpu.make_async_copy(h2.at[idx_a.at[j]], ra.at[b], sems[b][0]).wait()
            pltpu.make_async_copy(h2.at[idx_b.at[j]], rb.at[b], sems[b][1]).wait()

        def mulstore(j, b):
            def mulrow(i, carry2):
                for k in range(H // 16):
                    sl = pl.ds(k * 16, 16)
                    ra[b, i, sl] = ra[b, i, sl] * rb[b, i, sl]
                return carry2

            lax.fori_loop(0, KQ, mulrow, 0)
            pltpu.sync_copy(ra.at[b], p_out.at[pl.ds(obase + j * KQ, KQ)])

        start(0, 0)

        def pair(m, carry):
            j0 = m * 2
            start(j0 + 1, 1)
            wait(j0, 0)
            mulstore(j0, 0)

            @pl.when(j0 + 2 < NCHQ)
            def _():
                start(j0 + 2, 0)

            wait(j0 + 1, 1)
            mulstore(j0 + 1, 1)
            return carry

        lax.fori_loop(0, NCHQ // 2, pair, 0)

    @pl.when(c == 0)
    def _():
        run(h2_lo)

    @pl.when(c == 1)
    def _():
        run(h2_hi)


def _sc_queryprod(h2, qsr, qtr):
    f = pl.kernel(
        _qp_body,
        out_type=jax.ShapeDtypeStruct((NC * Q, H), jnp.float32),
        mesh=_vmesh(),
        scratch_types=[
            pltpu.VMEM((NCHQ, KQ), jnp.int32),
            pltpu.VMEM((NCHQ, KQ), jnp.int32),
            pltpu.VMEM((2, KQ, H), jnp.float32),
            pltpu.VMEM((2, KQ, H), jnp.float32),
            pltpu.SemaphoreType.DMA,
            pltpu.SemaphoreType.DMA,
            pltpu.SemaphoreType.DMA,
            pltpu.SemaphoreType.DMA,
        ],
    )
    return f(h2[0], h2[1], qsr, qtr)


RB = 1000
QB = 512


def _mm1_body(h_ref, w_ref, o_ref):
    m = jnp.dot(h_ref[:], w_ref[:], preferred_element_type=jnp.float32)
    o_ref[0] = m[:, :H]
    o_ref[1] = m[:, H:]


def _tc_mm1(h0, W1):
    return pl.pallas_call(
        _mm1_body,
        grid=(N // RB,),
        in_specs=[
            pl.BlockSpec((RB, D), lambda i: (i, 0)),
            pl.BlockSpec((D, D), lambda i: (0, 0)),
        ],
        out_specs=pl.BlockSpec((2, RB, H), lambda i: (0, i, 0)),
        out_shape=jax.ShapeDtypeStruct((2, N, H), jnp.float32),
    )(h0, W1)


def _scale_body(m_ref, d0_ref, d1_ref, o_ref):
    dinv = lax.rsqrt(1.0 + d0_ref[:, :1] + d1_ref[:, :1])
    o_ref[0] = m_ref[0] * dinv
    o_ref[1] = m_ref[1] * dinv


def _tc_scale(m, d0, d1):
    return pl.pallas_call(
        _scale_body,
        grid=(N // RB,),
        in_specs=[
            pl.BlockSpec((2, RB, H), lambda i: (0, i, 0)),
            pl.BlockSpec((RB, DW), lambda i: (i, 0)),
            pl.BlockSpec((RB, DW), lambda i: (i, 0)),
        ],
        out_specs=pl.BlockSpec((2, RB, H), lambda i: (0, i, 0)),
        out_shape=jax.ShapeDtypeStruct((2, N, H), jnp.float32),
    )(m, d0, d1)


def _pmm2_body(a_ref, hs_ref, d0_ref, d1_ref, b1_ref, g_ref, be_ref, w2_ref, o_ref):
    dinv = lax.rsqrt(1.0 + d0_ref[:, :1] + d1_ref[:, :1])
    lo = (a_ref[0] + hs_ref[0]) * dinv + b1_ref[:, :H]
    hi = (a_ref[1] + hs_ref[1]) * dinv + b1_ref[:, H:]
    lo = jnp.maximum(lo * BNS * g_ref[:, :H] + be_ref[:, :H], 0.0)
    hi = jnp.maximum(hi * BNS * g_ref[:, H:] + be_ref[:, H:], 0.0)
    m = (jnp.dot(lo, w2_ref[:H, :], preferred_element_type=jnp.float32)
         + jnp.dot(hi, w2_ref[H:, :], preferred_element_type=jnp.float32)) * dinv
    o_ref[0] = m[:, :H]
    o_ref[1] = m[:, H:]


def _tc_pmm2(agg, hs, d0, d1, b1r, gr, ber, W2):
    return pl.pallas_call(
        _pmm2_body,
        grid=(N // RB,),
        in_specs=[
            pl.BlockSpec((2, RB, H), lambda i: (0, i, 0)),
            pl.BlockSpec((2, RB, H), lambda i: (0, i, 0)),
            pl.BlockSpec((RB, DW), lambda i: (i, 0)),
            pl.BlockSpec((RB, DW), lambda i: (i, 0)),
            pl.BlockSpec((1, D), lambda i: (0, 0)),
            pl.BlockSpec((1, D), lambda i: (0, 0)),
            pl.BlockSpec((1, D), lambda i: (0, 0)),
            pl.BlockSpec((D, D), lambda i: (0, 0)),
        ],
        out_specs=pl.BlockSpec((2, RB, H), lambda i: (0, i, 0)),
        out_shape=jax.ShapeDtypeStruct((2, N, H), jnp.float32),
    )(agg, hs, d0, d1, b1r, gr, ber, W2)


def _post2_body(a_ref, hs_ref, d0_ref, d1_ref, b2_ref, o_ref):
    dinv = lax.rsqrt(1.0 + d0_ref[:, :1] + d1_ref[:, :1])
    o_ref[0] = (a_ref[0] + hs_ref[0]) * dinv + b2_ref[:, :H]
    o_ref[1] = (a_ref[1] + hs_ref[1]) * dinv + b2_ref[:, H:]


def _tc_post2(agg, hs, d0, d1, b2r):
    return pl.pallas_call(
        _post2_body,
        grid=(N // RB,),
        in_specs=[
            pl.BlockSpec((2, RB, H), lambda i: (0, i, 0)),
            pl.BlockSpec((2, RB, H), lambda i: (0, i, 0)),
            pl.BlockSpec((RB, DW), lambda i: (i, 0)),
            pl.BlockSpec((RB, DW), lambda i: (i, 0)),
            pl.BlockSpec((1, D), lambda i: (0, 0)),
        ],
        out_specs=pl.BlockSpec((2, RB, H), lambda i: (0, i, 0)),
        out_shape=jax.ShapeDtypeStruct((2, N, H), jnp.float32),
    )(agg, hs, d0, d1, b2r)


def _pred_body(p_ref, w1_ref, b1_ref, w2r_ref, b2_ref, o_ref):
    w1b = w1_ref[:].astype(jnp.bfloat16)
    z = (jnp.dot(p_ref[0].astype(jnp.bfloat16), w1b[:H, :],
                 preferred_element_type=jnp.float32)
         + jnp.dot(p_ref[1].astype(jnp.bfloat16), w1b[H:, :],
                   preferred_element_type=jnp.float32)
         + b1_ref[:])
    z = jnp.maximum(z, 0.0)
    t = jnp.sum(z * w2r_ref[:], axis=1, keepdims=True) + b2_ref[:]
    o_ref[:] = jax.nn.sigmoid(t)


def _tc_pred(p, pw1, pb1r, pw2r, pb2r):
    return pl.pallas_call(
        _pred_body,
        grid=(Q // QB,),
        in_specs=[
            pl.BlockSpec((2, QB, H), lambda i: (0, i, 0)),
            pl.BlockSpec((D, D), lambda i: (0, 0)),
            pl.BlockSpec((1, D), lambda i: (0, 0)),
            pl.BlockSpec((1, D), lambda i: (0, 0)),
            pl.BlockSpec((1, 1), lambda i: (0, 0)),
        ],
        out_specs=pl.BlockSpec((QB, 1), lambda i: (i, 0)),
        out_shape=jax.ShapeDtypeStruct((Q, 1), jnp.float32),
    )(p, pw1, pb1r, pw2r, pb2r)


def kernel(x, edge_index, edges, emb, W1, b1, W2, b2, bn_gamma, bn_beta, pw1, pb1, pw2, pb2):
    h0 = jnp.concatenate([emb, x], axis=1)
    src = edge_index[0]
    dst = edge_index[1]
    srcr = src.reshape(NS, NCHA, KA)
    dstr = dst.reshape(NS, NCHA, KA)
    dstr_deg = dst.reshape(NC * NS, NCHD, KD)
    qsr = edges[0].reshape(NS, NCHQ, KQ)
    qtr = edges[1].reshape(NS, NCHQ, KQ)

    deg = _sc_degree(dstr_deg)
    d0, d1 = deg[0], deg[1]

    m1 = _tc_mm1(h0, W1)
    hs1 = _tc_scale(m1, d0, d1)
    agg1 = _sc_agg(hs1, srcr, dstr).reshape(NC, N, H)

    hs2 = _tc_pmm2(agg1, hs1, d0, d1,
                   b1.reshape(1, D), bn_gamma.reshape(1, D), bn_beta.reshape(1, D), W2)
    agg2 = _sc_agg(hs2, srcr, dstr).reshape(NC, N, H)

    h2 = _tc_post2(agg2, hs2, d0, d1, b2.reshape(1, D))
    p = _sc_queryprod(h2, qsr, qtr).reshape(NC, Q, H)

    out = _tc_pred(p, pw1, pb1.reshape(1, D), pw2.reshape(1, D), pb2.reshape(1, 1))
    return out.reshape(-1)


# speedup vs baseline: 1.0015x; 1.0015x over previous
import jax
import jax.numpy as jnp
import numpy as np
from jax import lax
from jax.experimental import pallas as pl
from jax.experimental.pallas import tpu as pltpu
from jax.experimental.pallas import tpu_sc as plsc

N = 10000
D = 256
H = 128
E = 320000
Q = 65536
NC = 2
NS = 16
BNS = 1.0 / (1.0 + 1e-5) ** 0.5

RPT = 624

KD = 125
NCHD = E // (NC * NS) // KD
DW = 128
KA = 125
NCHA = E // NS // KA
PCH = 40
KQ = 128
NCHQ = Q // NS // KQ


def _per_tile_rows(s, fn):
    fn(s * RPT, RPT)

    @pl.when(s == 0)
    def _():
        fn(NS * RPT, N - NS * RPT)


def _vmesh():
    return plsc.VectorSubcoreMesh(
        core_axis_name="c", subcore_axis_name="s", num_cores=NC, num_subcores=NS)


def _deg_body(dstr_hbm, ones_hbm, zeros_hbm, deg_out, dacc, idx_v, ones_v):
    c = lax.axis_index("c")
    s = lax.axis_index("s")
    wid = c * NS + s
    _per_tile_rows(s, lambda o, n: pltpu.sync_copy(
        zeros_hbm.at[pl.ds(o, n)], dacc.at[pl.ds(o, n)]))
    pltpu.sync_copy(dstr_hbm.at[wid], idx_v)
    pltpu.sync_copy(ones_hbm, ones_v)
    plsc.subcore_barrier()

    def chunk(j, carry):
        pltpu.sync_copy(ones_v, dacc.at[idx_v.at[j]], add=True)
        return carry

    lax.fori_loop(0, NCHD, chunk, 0)
    plsc.subcore_barrier()
    _per_tile_rows(s, lambda o, n: pltpu.sync_copy(
        dacc.at[pl.ds(o, n)], deg_out.at[c, pl.ds(o, n)]))


def _sc_degree(dstr):
    f = pl.kernel(
        _deg_body,
        out_type=jax.ShapeDtypeStruct((NC, N, DW), jnp.float32),
        mesh=_vmesh(),
        scratch_types=[
            pltpu.VMEM_SHARED((N, DW), jnp.float32),
            pltpu.VMEM((NCHD, KD), jnp.int32),
            pltpu.VMEM((KD, DW), jnp.float32),
        ],
    )
    return f(dstr, jnp.ones((KD, DW), jnp.float32), jnp.zeros((N, DW), jnp.float32))


def _agg_body(hs_lo, hs_hi, srcr, dstr, agg_out, acc, idx_s, idx_d, rows,
              sem0, sem1):
    c = lax.axis_index("c")
    s = lax.axis_index("s")
    coff = c * N

    def run(hs_hbm):
        _per_tile_rows(s, lambda o, n: pltpu.sync_copy(
            hs_hbm.at[pl.ds(o, n)], acc.at[pl.ds(o, n)]))
        plsc.subcore_barrier()

        def start(j, b, sem):
            pltpu.async_copy(hs_hbm.at[idx_s.at[j]], rows.at[b], sem)

        def wait(j, b, sem):
            pltpu.make_async_copy(hs_hbm.at[idx_s.at[j]], rows.at[b], sem).wait()

        def scat(j, b):
            pltpu.sync_copy(rows.at[b], acc.at[idx_d.at[j]], add=True)

        def phase(p, carry):
            pltpu.sync_copy(srcr.at[s, pl.ds(p * PCH, PCH)], idx_s)
            pltpu.sync_copy(dstr.at[s, pl.ds(p * PCH, PCH)], idx_d)
            start(0, 0, sem0)

            def pair(m, carry2):
                j0 = m * 2
                start(j0 + 1, 1, sem1)
                wait(j0, 0, sem0)
                scat(j0, 0)

                @pl.when(j0 + 2 < PCH)
                def _():
                    start(j0 + 2, 0, sem0)

                wait(j0 + 1, 1, sem1)
                scat(j0 + 1, 1)
                return carry2

            lax.fori_loop(0, PCH // 2, pair, 0)
            return carry

        lax.fori_loop(0, NCHA // PCH, phase, 0)
        plsc.subcore_barrier()
        _per_tile_rows(s, lambda o, n: pltpu.sync_copy(
            acc.at[pl.ds(o, n)], agg_out.at[pl.ds(coff + o, n)]))

    @pl.when(c == 0)
    def _():
        run(hs_lo)

    @pl.when(c == 1)
    def _():
        run(hs_hi)


def _sc_agg(hs, srcr, dstr):
    f = pl.kernel(
        _agg_body,
        out_type=jax.ShapeDtypeStruct((NC * N, H), jnp.float32),
        mesh=_vmesh(),
        scratch_types=[
            pltpu.VMEM_SHARED((N, H), jnp.float32),
            pltpu.VMEM((PCH, KA), jnp.int32),
            pltpu.VMEM((PCH, KA), jnp.int32),
            pltpu.VMEM((2, KA, H), jnp.float32),
            pltpu.SemaphoreType.DMA,
            pltpu.SemaphoreType.DMA,
        ],
    )
    return f(hs[0], hs[1], srcr, dstr)


def _qp_body(h2_lo, h2_hi, qsr, qtr, p_out, idx_a, idx_b, ra, rb,
             sa0, sb0, sa1, sb1):
    c = lax.axis_index("c")
    s = lax.axis_index("s")
    pltpu.sync_copy(qsr.at[s], idx_a)
    pltpu.sync_copy(qtr.at[s], idx_b)
    obase = c * Q + s * (Q // NS)

    def run(h2):
        sems = ((sa0, sb0), (sa1, sb1))

        def start(j, b):
            pltpu.async_copy(h2.at[idx_a.at[j]], ra.at[b], sems[b][0])
            pltpu.async_copy(h2.at[idx_b.at[j]], rb.at[b], sems[b][1])

        def wait(j, b):
            pltpu.make_async_copy(h2.at[idx_a.at[j]], ra.at[b], sems[b][0]).wait()
            pltpu.make_async_copy(h2.at[idx_b.at[j]], rb.at[b], sems[b][1]).wait()

        def mulstore(j, b):
            def mulrow(i, carry2):
                for k in range(H // 16):
                    sl = pl.ds(k * 16, 16)
                    ra[b, i, sl] = ra[b, i, sl] * rb[b, i, sl]
                return carry2

            lax.fori_loop(0, KQ, mulrow, 0)
            pltpu.sync_copy(ra.at[b], p_out.at[pl.ds(obase + j * KQ, KQ)])

        start(0, 0)

        def pair(m, carry):
            j0 = m * 2
            start(j0 + 1, 1)
            wait(j0, 0)
            mulstore(j0, 0)

            @pl.when(j0 + 2 < NCHQ)
            def _():
                start(j0 + 2, 0)

            wait(j0 + 1, 1)
            mulstore(j0 + 1, 1)
            return carry

        lax.fori_loop(0, NCHQ // 2, pair, 0)

    @pl.when(c == 0)
    def _():
        run(h2_lo)

    @pl.when(c == 1)
    def _():
        run(h2_hi)


def _sc_queryprod(h2, qsr, qtr):
    f = pl.kernel(
        _qp_body,
        out_type=jax.ShapeDtypeStruct((NC * Q, H), jnp.float32),
        mesh=_vmesh(),
        scratch_types=[
            pltpu.VMEM((NCHQ, KQ), jnp.int32),
            pltpu.VMEM((NCHQ, KQ), jnp.int32),
            pltpu.VMEM((2, KQ, H), jnp.float32),
            pltpu.VMEM((2, KQ, H), jnp.float32),
            pltpu.SemaphoreType.DMA,
            pltpu.SemaphoreType.DMA,
            pltpu.SemaphoreType.DMA,
            pltpu.SemaphoreType.DMA,
        ],
    )
    return f(h2[0], h2[1], qsr, qtr)


RB = 1000
QB = 512


def _mm1_body(h_ref, w_ref, o_ref):
    m = jnp.dot(h_ref[:], w_ref[:], preferred_element_type=jnp.float32)
    o_ref[0] = m[:, :H]
    o_ref[1] = m[:, H:]


def _tc_mm1(h0, W1):
    return pl.pallas_call(
        _mm1_body,
        grid=(N // RB,),
        in_specs=[
            pl.BlockSpec((RB, D), lambda i: (i, 0)),
            pl.BlockSpec((D, D), lambda i: (0, 0)),
        ],
        out_specs=pl.BlockSpec((2, RB, H), lambda i: (0, i, 0)),
        out_shape=jax.ShapeDtypeStruct((2, N, H), jnp.float32),
    )(h0, W1)


def _scale_body(m_ref, d0_ref, d1_ref, o_ref):
    dinv = lax.rsqrt(1.0 + d0_ref[:, :1] + d1_ref[:, :1])
    o_ref[0] = m_ref[0] * dinv
    o_ref[1] = m_ref[1] * dinv


def _tc_scale(m, d0, d1):
    return pl.pallas_call(
        _scale_body,
        grid=(N // RB,),
        in_specs=[
            pl.BlockSpec((2, RB, H), lambda i: (0, i, 0)),
            pl.BlockSpec((RB, DW), lambda i: (i, 0)),
            pl.BlockSpec((RB, DW), lambda i: (i, 0)),
        ],
        out_specs=pl.BlockSpec((2, RB, H), lambda i: (0, i, 0)),
        out_shape=jax.ShapeDtypeStruct((2, N, H), jnp.float32),
    )(m, d0, d1)


def _pmm2_body(a_ref, hs_ref, d0_ref, d1_ref, b1_ref, g_ref, be_ref, w2_ref, o_ref):
    dinv = lax.rsqrt(1.0 + d0_ref[:, :1] + d1_ref[:, :1])
    lo = (a_ref[0] + hs_ref[0]) * dinv + b1_ref[:, :H]
    hi = (a_ref[1] + hs_ref[1]) * dinv + b1_ref[:, H:]
    lo = jnp.maximum(lo * BNS * g_ref[:, :H] + be_ref[:, :H], 0.0)
    hi = jnp.maximum(hi * BNS * g_ref[:, H:] + be_ref[:, H:], 0.0)
    m = (jnp.dot(lo, w2_ref[:H, :], preferred_element_type=jnp.float32)
         + jnp.dot(hi, w2_ref[H:, :], preferred_element_type=jnp.float32)) * dinv
    o_ref[0] = m[:, :H]
    o_ref[1] = m[:, H:]


def _tc_pmm2(agg, hs, d0, d1, b1r, gr, ber, W2):
    return pl.pallas_call(
        _pmm2_body,
        grid=(N // RB,),
        in_specs=[
            pl.BlockSpec((2, RB, H), lambda i: (0, i, 0)),
            pl.BlockSpec((2, RB, H), lambda i: (0, i, 0)),
            pl.BlockSpec((RB, DW), lambda i: (i, 0)),
            pl.BlockSpec((RB, DW), lambda i: (i, 0)),
            pl.BlockSpec((1, D), lambda i: (0, 0)),
            pl.BlockSpec((1, D), lambda i: (0, 0)),
            pl.BlockSpec((1, D), lambda i: (0, 0)),
            pl.BlockSpec((D, D), lambda i: (0, 0)),
        ],
        out_specs=pl.BlockSpec((2, RB, H), lambda i: (0, i, 0)),
        out_shape=jax.ShapeDtypeStruct((2, N, H), jnp.float32),
    )(agg, hs, d0, d1, b1r, gr, ber, W2)


def _post2_body(a_ref, hs_ref, d0_ref, d1_ref, b2_ref, o_ref):
    dinv = lax.rsqrt(1.0 + d0_ref[:, :1] + d1_ref[:, :1])
    o_ref[0] = (a_ref[0] + hs_ref[0]) * dinv + b2_ref[:, :H]
    o_ref[1] = (a_ref[1] + hs_ref[1]) * dinv + b2_ref[:, H:]


def _tc_post2(agg, hs, d0, d1, b2r):
    return pl.pallas_call(
        _post2_body,
        grid=(N // RB,),
        in_specs=[
            pl.BlockSpec((2, RB, H), lambda i: (0, i, 0)),
            pl.BlockSpec((2, RB, H), lambda i: (0, i, 0)),
            pl.BlockSpec((RB, DW), lambda i: (i, 0)),
            pl.BlockSpec((RB, DW), lambda i: (i, 0)),
            pl.BlockSpec((1, D), lambda i: (0, 0)),
        ],
        out_specs=pl.BlockSpec((2, RB, H), lambda i: (0, i, 0)),
        out_shape=jax.ShapeDtypeStruct((2, N, H), jnp.float32),
    )(agg, hs, d0, d1, b2r)


def _pred_body(p_ref, w1_ref, b1_ref, w2r_ref, b2_ref, o_ref):
    w1b = w1_ref[:].astype(jnp.bfloat16)
    z = (jnp.dot(p_ref[0].astype(jnp.bfloat16), w1b[:H, :],
                 preferred_element_type=jnp.float32)
         + jnp.dot(p_ref[1].astype(jnp.bfloat16), w1b[H:, :],
                   preferred_element_type=jnp.float32)
         + b1_ref[:])
    z = jnp.maximum(z, 0.0)
    t = jnp.sum(z * w2r_ref[:], axis=1, keepdims=True) + b2_ref[:]
    o_ref[:] = jax.nn.sigmoid(t)


def _tc_pred(p, pw1, pb1r, pw2r, pb2r):
    return pl.pallas_call(
        _pred_body,
        grid=(Q // QB,),
        in_specs=[
            pl.BlockSpec((2, QB, H), lambda i: (0, i, 0)),
            pl.BlockSpec((D, D), lambda i: (0, 0)),
            pl.BlockSpec((1, D), lambda i: (0, 0)),
            pl.BlockSpec((1, D), lambda i: (0, 0)),
            pl.BlockSpec((1, 1), lambda i: (0, 0)),
        ],
        out_specs=pl.BlockSpec((QB, 1), lambda i: (i, 0)),
        out_shape=jax.ShapeDtypeStruct((Q, 1), jnp.float32),
    )(p, pw1, pb1r, pw2r, pb2r)


def kernel(x, edge_index, edges, emb, W1, b1, W2, b2, bn_gamma, bn_beta, pw1, pb1, pw2, pb2):
    h0 = jnp.concatenate([emb, x], axis=1)
    src = edge_index[0]
    dst = edge_index[1]
    srcr = src.reshape(NS, NCHA, KA)
    dstr = dst.reshape(NS, NCHA, KA)
    dstr_deg = dst.reshape(NC * NS, NCHD, KD)
    qsr = edges[0].reshape(NS, NCHQ, KQ)
    qtr = edges[1].reshape(NS, NCHQ, KQ)

    deg = _sc_degree(dstr_deg)
    d0, d1 = deg[0], deg[1]

    m1 = _tc_mm1(h0, W1)
    hs1 = _tc_scale(m1, d0, d1)
    agg1 = _sc_agg(hs1, srcr, dstr).reshape(NC, N, H)

    hs2 = _tc_pmm2(agg1, hs1, d0, d1,
                   b1.reshape(1, D), bn_gamma.reshape(1, D), bn_beta.reshape(1, D), W2)
    agg2 = _sc_agg(hs2, srcr, dstr).reshape(NC, N, H)

    h2 = _tc_post2(agg2, hs2, d0, d1, b2.reshape(1, D))
    p = _sc_queryprod(h2, qsr, qtr).reshape(NC, Q, H)

    out = _tc_pred(p, pw1, pb1.reshape(1, D), pw2.reshape(1, D), pb2.reshape(1, 1))
    return out.reshape(-1)
